# baseline (speedup 1.0000x reference)
import functools
import math

import jax
import jax.numpy as jnp
from jax import lax
from jax.experimental import pallas as pl
from jax.experimental.pallas import tpu as pltpu
from jax.experimental.pallas import tpu_sc as plsc

_B, _TPH, _TMEL, _H = 16, 1024, 2048, 256
_F0_BIN = 256
_F0_MIN, _F0_MAX = 50.0, 1100.0
_MEL_MIN = 1127.0 * math.log(1.0 + _F0_MIN / 700.0)
_MEL_MAX = 1127.0 * math.log(1.0 + _F0_MAX / 700.0)

_HROWS = _B * _TPH
_ROWS = _B * _TMEL

_NSPLIT = 4
_PB = _B // _NSPLIT
_PROWS = _PB * _TMEL

_NC, _NS, _L = 2, 16, 16
_NW = _NC * _NS
_RPW = _PROWS // _NW
_CH = 128
_NCH = _RPW // _CH


def _prep_body(mel2ph_ref, gidx_ref):
    m = mel2ph_ref[...]
    b = lax.broadcasted_iota(jnp.int32, m.shape, 0)
    gidx_ref[...] = b * _TPH + jnp.maximum(m - 1, 0)


def _make_sc_body(piece):
    def _sc_body(hub_ref, gidx_ref, out_ref, gi_all, a0, a1,
                 gsem, wsem0, wsem1):
        wid = lax.axis_index("s") * _NC + lax.axis_index("c")
        base = wid * _RPW
        pltpu.sync_copy(gidx_ref.at[pl.ds(piece * _PROWS + base, _RPW)],
                        gi_all)
        bufs, wsems = (a0, a1), (wsem0, wsem1)
        writes = [None, None]
        for i in range(_NCH):
            b = i % 2
            if writes[b] is not None:
                writes[b].wait()
            pltpu.async_copy(
                hub_ref.at[gi_all.at[pl.ds(i * _CH, _CH)]], bufs[b],
                gsem).wait()
            writes[b] = pltpu.async_copy(
                bufs[b], out_ref.at[pl.ds(base + i * _CH, _CH)], wsems[b])
        for b in range(min(2, _NCH)):
            if writes[b] is not None:
                writes[b].wait()

    return _sc_body


@functools.lru_cache(maxsize=None)
def _get_sc_call(piece):
    return pl.kernel(
        _make_sc_body(piece),
        out_type=jax.ShapeDtypeStruct((_PROWS, _H), jnp.float32),
        mesh=plsc.VectorSubcoreMesh(core_axis_name="c", subcore_axis_name="s"),
        scratch_types=[
            pltpu.VMEM((_RPW,), jnp.int32),
            pltpu.VMEM((_CH, _H), jnp.float32),
            pltpu.VMEM((_CH, _H), jnp.float32),
            pltpu.SemaphoreType.DMA,
            pltpu.SemaphoreType.DMA,
            pltpu.SemaphoreType.DMA,
        ],
    )


def _epi_math(m, f0):
    f0d = jnp.where(m == 0, 0.0, jnp.exp2(f0))
    f0_mel = 1127.0 * jnp.log(1.0 + f0d / 700.0)
    f0_mel = jnp.where(
        f0_mel > 0,
        (f0_mel - _MEL_MIN) * (_F0_BIN - 2) / (_MEL_MAX - _MEL_MIN) + 1.0,
        f0_mel)
    f0_mel = jnp.where(f0_mel <= 1.0, 1.0, f0_mel)
    f0_mel = jnp.where(f0_mel > _F0_BIN - 1, float(_F0_BIN - 1), f0_mel)
    pitch = (f0_mel + 0.5).astype(jnp.int32)
    return f0d, pitch


def _finish_first(dec_ref, mel_ref, f0_ref, pe_ref, spk_ref, out_ref, f0d_ref):
    _finish_common(dec_ref, mel_ref, f0_ref, pe_ref, spk_ref, out_ref, f0d_ref)


def _finish_next(dec_ref, mel_ref, f0_ref, pe_ref, spk_ref,
                 out_prev, f0d_prev, out_ref, f0d_ref):
    _finish_common(dec_ref, mel_ref, f0_ref, pe_ref, spk_ref, out_ref, f0d_ref)


def _finish_common(dec_ref, mel_ref, f0_ref, pe_ref, spk_ref, out_ref, f0d_ref):
    m = mel_ref[0]
    f0d, pitch = _epi_math(m, f0_ref[0])
    f0d_ref[0] = f0d
    onehot = (lax.broadcasted_iota(jnp.int32, (_F0_BIN, _TMEL), 0)
              == pitch).astype(jnp.float32)
    pitch_t = lax.dot_general(pe_ref[...], onehot, (((0,), (0,)), ((), ())),
                              preferred_element_type=jnp.float32)
    x = dec_ref[...]
    spk = spk_ref[0]
    mask = (m > 0).astype(jnp.float32)
    out_ref[0] = (jnp.transpose(x + spk, (1, 0)) + pitch_t) * mask


def _epi_call(piece, dec, mel3, f03, pe, spk3, prev):
    batch0 = piece * _PB
    in_specs = [
        pl.BlockSpec((_TMEL, _H), lambda b: (b, 0)),
        pl.BlockSpec((1, 1, _TMEL), lambda b: (b + batch0, 0, 0)),
        pl.BlockSpec((1, 1, _TMEL), lambda b: (b + batch0, 0, 0)),
        pl.BlockSpec((_F0_BIN, _H), lambda b: (0, 0)),
        pl.BlockSpec((1, 1, _H), lambda b: (b + batch0, 0, 0)),
    ]
    out_specs = (
        pl.BlockSpec((1, _H, _TMEL), lambda b: (b + batch0, 0, 0)),
        pl.BlockSpec((1, 1, _TMEL), lambda b: (b + batch0, 0, 0)),
    )
    out_shape = (
        jax.ShapeDtypeStruct((_B, _H, _TMEL), jnp.float32),
        jax.ShapeDtypeStruct((_B, 1, _TMEL), jnp.float32),
    )
    if prev is None:
        return pl.pallas_call(
            _finish_first, grid=(_PB,), in_specs=in_specs,
            out_specs=out_specs, out_shape=out_shape,
        )(dec, mel3, f03, pe, spk3)
    in_specs = in_specs + [
        pl.BlockSpec(memory_space=pl.ANY),
        pl.BlockSpec(memory_space=pl.ANY),
    ]
    return pl.pallas_call(
        _finish_next, grid=(_PB,), in_specs=in_specs,
        out_specs=out_specs, out_shape=out_shape,
        input_output_aliases={5: 0, 6: 1},
    )(dec, mel3, f03, pe, spk3, prev[0], prev[1])


def kernel(hubert, spk_embed, f0, pitch_embed, mel2ph):
    hub = hubert.reshape(_HROWS, _H)
    gidx = pl.pallas_call(
        _prep_body,
        out_shape=jax.ShapeDtypeStruct((_B, _TMEL), jnp.int32),
    )(mel2ph).reshape(_ROWS)

    decs = [_get_sc_call(p)(hub, gidx) for p in range(_NSPLIT)]

    mel3 = mel2ph.reshape(_B, 1, _TMEL)
    f03 = f0.reshape(_B, 1, _TMEL)
    spk3 = spk_embed.reshape(_B, 1, _H)
    pe = pitch_embed[:_F0_BIN]

    prev = None
    for p in range(_NSPLIT):
        prev = _epi_call(p, decs[p], mel3, f03, pe, spk3, prev)

    out, f0d = prev
    return out, f0d.reshape(_B, _TMEL)


# speedup vs baseline: 1.1245x; 1.1245x over previous
import functools
import math

import jax
import jax.numpy as jnp
from jax import lax
from jax.experimental import pallas as pl
from jax.experimental.pallas import tpu as pltpu
from jax.experimental.pallas import tpu_sc as plsc

_B, _TPH, _TMEL, _H = 16, 1024, 2048, 256
_HP = _H // 2
_F0_BIN = 256
_F0_MIN, _F0_MAX = 50.0, 1100.0
_MEL_MIN = 1127.0 * math.log(1.0 + _F0_MIN / 700.0)
_MEL_MAX = 1127.0 * math.log(1.0 + _F0_MAX / 700.0)

_HROWS = _B * _TPH
_ROWS = _B * _TMEL

_NC, _NS, _L = 2, 16, 16
_NW = _NC * _NS
_RPW = _ROWS // _NW
_CH = 128
_NCH = _RPW // _CH


def _prep_body(mel2ph_ref, gidx_ref):
    m = mel2ph_ref[...]
    b = lax.broadcasted_iota(jnp.int32, m.shape, 0)
    gidx_ref[...] = b * _TPH + jnp.maximum(m - 1, 0)


def _sc_body(hub_ref, gidx_ref, out_ref, gi_all, a0, a1, gsem, wsem0, wsem1):
    wid = lax.axis_index("s") * _NC + lax.axis_index("c")
    base = wid * _RPW
    pltpu.sync_copy(gidx_ref.at[pl.ds(base, _RPW)], gi_all)
    bufs, wsems = (a0, a1), (wsem0, wsem1)
    writes = [None, None]
    for i in range(_NCH):
        b = i % 2
        if writes[b] is not None:
            writes[b].wait()
        pltpu.async_copy(
            hub_ref.at[gi_all.at[pl.ds(i * _CH, _CH)]], bufs[b], gsem).wait()
        writes[b] = pltpu.async_copy(
            bufs[b], out_ref.at[pl.ds(base + i * _CH, _CH)], wsems[b])
    writes[0].wait()
    writes[1].wait()


@functools.lru_cache(maxsize=None)
def _get_sc_call():
    return pl.kernel(
        _sc_body,
        out_type=jax.ShapeDtypeStruct((_ROWS, _HP), jnp.int32),
        mesh=plsc.VectorSubcoreMesh(core_axis_name="c", subcore_axis_name="s"),
        scratch_types=[
            pltpu.VMEM((_RPW,), jnp.int32),
            pltpu.VMEM((_CH, _HP), jnp.int32),
            pltpu.VMEM((_CH, _HP), jnp.int32),
            pltpu.SemaphoreType.DMA,
            pltpu.SemaphoreType.DMA,
            pltpu.SemaphoreType.DMA,
        ],
    )


def _finish_body(dec_ref, mel_ref, f0_ref, pe_ref, spk_ref, out_ref, f0d_ref):
    m = mel_ref[0]
    f0 = f0_ref[0]
    f0d = jnp.where(m == 0, 0.0, jnp.exp2(f0))
    f0d_ref[0] = f0d
    f0_mel = 1127.0 * jnp.log(1.0 + f0d / 700.0)
    f0_mel = jnp.where(
        f0_mel > 0,
        (f0_mel - _MEL_MIN) * (_F0_BIN - 2) / (_MEL_MAX - _MEL_MIN) + 1.0,
        f0_mel)
    f0_mel = jnp.where(f0_mel <= 1.0, 1.0, f0_mel)
    f0_mel = jnp.where(f0_mel > _F0_BIN - 1, float(_F0_BIN - 1), f0_mel)
    pitch = (f0_mel + 0.5).astype(jnp.int32)
    onehot = (lax.broadcasted_iota(jnp.int32, (_F0_BIN, _TMEL), 0)
              == pitch).astype(jnp.float32)
    pitch_t = lax.dot_general(pe_ref[...], onehot, (((0,), (0,)), ((), ())),
                              preferred_element_type=jnp.float32)
    xp = dec_ref[...]
    lo = lax.bitcast_convert_type(xp << 16, jnp.float32)
    hi = lax.bitcast_convert_type(xp & jnp.int32(-65536), jnp.float32)
    x = jnp.concatenate([lo, hi], axis=1)
    spk = spk_ref[0]
    mask = (m > 0).astype(jnp.float32)
    out_ref[0] = (jnp.transpose(x + spk, (1, 0)) + pitch_t) * mask


def kernel(hubert, spk_embed, f0, pitch_embed, mel2ph):
    hub16 = hubert.reshape(_HROWS, _H).astype(jnp.bfloat16)
    au = lax.bitcast_convert_type(hub16[:, :_HP], jnp.uint16).astype(jnp.uint32)
    bu = lax.bitcast_convert_type(hub16[:, _HP:], jnp.uint16).astype(jnp.uint32)
    hub_packed = lax.bitcast_convert_type(au | (bu << 16), jnp.int32)

    gidx = pl.pallas_call(
        _prep_body,
        out_shape=jax.ShapeDtypeStruct((_B, _TMEL), jnp.int32),
    )(mel2ph).reshape(_ROWS)

    dec = _get_sc_call()(hub_packed, gidx)

    out, f0d = pl.pallas_call(
        _finish_body,
        grid=(_B,),
        in_specs=[
            pl.BlockSpec((_TMEL, _HP), lambda b: (b, 0)),
            pl.BlockSpec((1, 1, _TMEL), lambda b: (b, 0, 0)),
            pl.BlockSpec((1, 1, _TMEL), lambda b: (b, 0, 0)),
            pl.BlockSpec((_F0_BIN, _H), lambda b: (0, 0)),
            pl.BlockSpec((1, 1, _H), lambda b: (b, 0, 0)),
        ],
        out_specs=(
            pl.BlockSpec((1, _H, _TMEL), lambda b: (b, 0, 0)),
            pl.BlockSpec((1, 1, _TMEL), lambda b: (b, 0, 0)),
        ),
        out_shape=(
            jax.ShapeDtypeStruct((_B, _H, _TMEL), jnp.float32),
            jax.ShapeDtypeStruct((_B, 1, _TMEL), jnp.float32),
        ),
    )(dec, mel2ph.reshape(_B, 1, _TMEL), f0.reshape(_B, 1, _TMEL),
      pitch_embed[:_F0_BIN], spk_embed.reshape(_B, 1, _H))

    return out, f0d.reshape(_B, _TMEL)
